# Initial kernel scaffold; baseline (speedup 1.0000x reference)
import jax
import jax.numpy as jnp
from jax.experimental import pallas as pl


def kernel(task_size, server_speeds, server_loads, network_conditions, server_distances, params):
    raise NotImplementedError("write your pallas kernel here")


# speedup vs baseline: 167.0062x; 167.0062x over previous
import jax
import jax.numpy as jnp
import numpy as np
from jax.experimental import pallas as pl

NS = 256
NV = 257
NP = 264
H = 128
NHEADS = 4
HD = 32
NL = 4
BD = 24
NEG = -1.0e30


def _lrelu(t):
    return jnp.where(t >= 0, t, 0.2 * t)


def _ln(t, g, b):
    mu = jnp.mean(t, axis=-1, keepdims=True)
    var = jnp.mean((t - mu) ** 2, axis=-1, keepdims=True)
    return (t - mu) / jnp.sqrt(var + 1e-5) * g + b


def _fwd_body(x0, neW, ne_b, ne_g, ne_be,
              W1d, W1s, b1, g1, be1, W2, b2,
              aWs, aWd, attb,
              gatW, Ad, AsT, gat_b,
              lng, lnbe,
              advW1, adv_b1, adv_g, adv_be, advW2, adv_b2,
              valW1, val_b1, val_g, val_be, valW2, val_b2,
              q_out, v_out):
    f32 = jnp.float32
    row_iota = jax.lax.broadcasted_iota(jnp.int32, (NP, 1), 0)
    row_mask = row_iota < NV

    X = _lrelu(jnp.dot(x0[:, :], neW[:, :], preferred_element_type=f32)
               + ne_b[:, :])
    X = _ln(X, ne_g[:, :], ne_be[:, :])
    X = jnp.where(row_mask, X, 0.0)

    col_iota2 = jax.lax.broadcasted_iota(jnp.int32, (NP, NP), 1)
    col_invalid = col_iota2 >= NV

    for l in range(NL):
        u = jnp.dot(X, W1d[l], preferred_element_type=f32) + b1[l]
        v = jnp.dot(X, W1s[l], preferred_element_type=f32)
        atts_ln = jnp.dot(X, aWs[l], preferred_element_type=f32) + attb[l]
        attd_ln = jnp.dot(X, aWd[l], preferred_element_type=f32)
        xg = jnp.dot(X, gatW[l], preferred_element_type=f32)

        ios3 = jax.lax.broadcasted_iota(jnp.int32, (NP, BD, H), 0)
        iod3 = jax.lax.broadcasted_iota(jnp.int32, (NP, BD, H), 1)
        conv_blocks = []
        for ci in range(NP // BD):
            d0 = ci * BD
            u_blk = u[d0:d0 + BD, :]
            h1 = _lrelu(v[:, None, :] + u_blk[None, :, :])
            h1 = _ln(h1, g1[l], be1[l])
            h2 = _lrelu(
                jnp.dot(h1.reshape(NP * BD, H), W2[l],
                        preferred_element_type=f32) + b2[l]).reshape(NP, BD, H)
            att3 = jax.nn.sigmoid(atts_ln[:, None, :]
                                  + attd_ln[d0:d0 + BD][None, :, :])
            msg = h2 * att3
            invalid = (ios3 >= NV) | (ios3 == (iod3 + d0))
            if l % 2 == 0:
                m = jnp.where(invalid, NEG, msg)
                conv_blk = jnp.max(m, axis=0)
            else:
                m = jnp.where(invalid, 0.0, msg)
                conv_blk = jnp.sum(m, axis=0) * (1.0 / 256.0)
            conv_blocks.append(conv_blk)
        x_conv = jnp.concatenate(conv_blocks, axis=0)

        adst = jnp.dot(xg, Ad[l], preferred_element_type=f32)
        outs = []
        for h in range(NHEADS):
            asrc_row = jax.lax.dot_general(
                AsT[l][h:h + 1, :], xg, (((1,), (1,)), ((), ())),
                preferred_element_type=f32)
            L = _lrelu(adst[:, h:h + 1] + asrc_row)
            L = jnp.where(col_invalid, NEG, L)
            mx = jnp.max(L, axis=-1, keepdims=True)
            E = jnp.exp(L - mx)
            den = jnp.sum(E, axis=-1, keepdims=True)
            P = E / (den + 1e-16)
            outs.append(jnp.dot(P, xg[:, h * HD:(h + 1) * HD],
                                preferred_element_type=f32))
        x_att = jnp.concatenate(outs, axis=1) + gat_b[l]

        Xn = X + x_conv + x_att
        Xn = _ln(Xn, lng[l], lnbe[l])
        Xn = _lrelu(Xn)
        X = jnp.where(row_mask, Xn, 0.0)

    t = X[0:1, :]
    a = _lrelu(jnp.dot(t, advW1[:, :], preferred_element_type=f32)
               + adv_b1[:, :])
    a = _ln(a, adv_g[:, :], adv_be[:, :])
    a2 = jnp.dot(a, advW2[:, :], preferred_element_type=f32) + adv_b2[:, :]
    vv = _lrelu(jnp.dot(t, valW1[:, :], preferred_element_type=f32)
                + val_b1[:, :])
    vv = _ln(vv, val_g[:, :], val_be[:, :])
    v2 = jnp.dot(vv, valW2[:, :], preferred_element_type=f32) + val_b2[:, :]
    q = v2 + (a2 - jnp.mean(a2, keepdims=True))
    q_out[:, :] = q
    v_out[:, :] = v2


def kernel(task_size, server_speeds, server_loads, network_conditions,
           server_distances, params):
    f32 = jnp.float32
    task_feat = jnp.concatenate([task_size, jnp.zeros((3,), f32)])[None, :]
    srv_feat = jnp.stack([server_speeds, server_loads, network_conditions,
                          server_distances], axis=1)
    x0 = jnp.concatenate([task_feat, srv_feat], axis=0)
    x0p = jnp.zeros((NP, H), f32).at[:NV, :4].set(x0)
    neWp = jnp.zeros((H, H), f32).at[:4, :].set(params['ne_W'])

    def stack(fmt, reshape=None):
        arrs = [params[fmt.format(l)] for l in range(NL)]
        if reshape is not None:
            arrs = [a.reshape(reshape) for a in arrs]
        return jnp.stack(arrs)

    W1d = stack('ec{}_W1')[:, :H, :]
    W1s = stack('ec{}_W1')[:, H:, :]
    b1 = stack('ec{}_b1', (1, H))
    g1 = stack('ec{}_g1', (1, H))
    be1 = stack('ec{}_be1', (1, H))
    W2 = stack('ec{}_W2')
    b2 = stack('ec{}_b2', (1, H))
    aWs = jnp.tile(stack('ec{}_attW')[:, H:, :], (1, 1, H))
    aWd = jnp.tile(stack('ec{}_attW')[:, :H, :], (1, 1, H))
    attb = stack('ec{}_attb', (1, 1))
    gatW = stack('gat{}_W')
    eye = jnp.eye(NHEADS, dtype=f32)
    def blockdiag(a):
        return (eye[:, None, :] * a[:, :, None]).reshape(H, NHEADS)
    Ad = jnp.stack([blockdiag(params[f'gat{l}_ad']) for l in range(NL)])
    AsT = jnp.stack([jnp.transpose(blockdiag(params[f'gat{l}_as']))
                     for l in range(NL)])
    gat_b = stack('gat{}_b', (1, H))
    lng = stack('ln{}_g', (1, H))
    lnbe = stack('ln{}_be', (1, H))

    out_shape = [jax.ShapeDtypeStruct((1, NS), f32),
                 jax.ShapeDtypeStruct((1, 1), f32)]
    q, v = pl.pallas_call(_fwd_body, out_shape=out_shape)(
        x0p, neWp, params['ne_b'].reshape(1, H), params['ne_g'].reshape(1, H),
        params['ne_be'].reshape(1, H),
        W1d, W1s, b1, g1, be1, W2, b2,
        aWs, aWd, attb,
        gatW, Ad, AsT, gat_b,
        lng, lnbe,
        params['adv_W1'], params['adv_b1'].reshape(1, H),
        params['adv_g'].reshape(1, H), params['adv_be'].reshape(1, H),
        params['adv_W2'], params['adv_b2'].reshape(1, NS),
        params['val_W1'], params['val_b1'].reshape(1, H),
        params['val_g'].reshape(1, H), params['val_be'].reshape(1, H),
        params['val_W2'], params['val_b2'].reshape(1, 1))
    return (q.reshape(NS), v.reshape(1))
